# baseline (speedup 1.0000x reference)
import functools

import jax
import jax.numpy as jnp
from jax import lax
from jax.experimental import pallas as pl
from jax.experimental.pallas import tpu as pltpu
from jax.experimental.pallas import tpu_sc as plsc

_B, _L, _V, _D = 4096, 200, 32, 129
_DM = _D - 1
_R = _B * _L
_NW = 32
_RPW = _R // _NW
_K = 128
_NCH = _RPW // _K


def _build_tables(token_table, ooba_table, pos_table):
    def body(tok_ref, ooba_ref, pos_ref, main_ref, tail_ref):
        pos = pos_ref[...]
        main_ref[...] = tok_ref[...][None, :, :] + pos[:, None, :_DM]
        tail_ref[...] = pos[:, _DM][:, None] + ooba_ref[...][:, 0][None, :]

    return pl.pallas_call(
        body,
        out_shape=(
            jax.ShapeDtypeStruct((_L, _V, _DM), jnp.float32),
            jax.ShapeDtypeStruct((_L, _V), jnp.float32),
        ),
    )(token_table, ooba_table, pos_table)


def _sc_gather(main, tail, xflat):
    mesh = plsc.VectorSubcoreMesh(core_axis_name="c", subcore_axis_name="s")

    @functools.partial(
        pl.kernel,
        mesh=mesh,
        out_type=jax.ShapeDtypeStruct((_R, _D), jnp.float32),
        scratch_types=[
            pltpu.VMEM((_L * _V,), jnp.float32),
            [pltpu.VMEM((_K,), jnp.int32)] * 2,
            [pltpu.VMEM((_K, 1), jnp.float32)] * 2,
            [pltpu.VMEM((_K, _DM), jnp.float32)] * 2,
            [pltpu.SemaphoreType.DMA] * 2,
        ],
        compiler_params=pltpu.CompilerParams(
            needs_layout_passes=False, use_tc_tiling_on_sc=False),
    )
    def k(main_hbm, tail_hbm, x_hbm, out_hbm, tail_v, idx_v, tails_v, rows_v, sem):
        wid = lax.axis_index("s") * 2 + lax.axis_index("c")
        base = wid * _RPW
        lane = lax.iota(jnp.int32, 16)
        zero16 = jnp.zeros((16,), jnp.int32)
        pltpu.sync_copy(tail_hbm, tail_v)

        def prep_start(g, p):
            r0 = base + g * _K
            pltpu.sync_copy(x_hbm.at[pl.ds(r0, _K)], idx_v[p])
            for j in range(_K // 16):
                xv = idx_v[p][pl.ds(j * 16, 16)]
                idx16 = lax.rem(r0 + j * 16 + lane, _L) * _V + xv
                idx_v[p][pl.ds(j * 16, 16)] = idx16
                tvals = plsc.load_gather(tail_v, [idx16])
                plsc.store_scatter(tails_v[p], [j * 16 + lane, zero16], tvals)
            pltpu.async_copy(main_hbm.at[idx_v[p]], rows_v[p], sem[p])

        def wait_store(g, p):
            r0 = base + g * _K
            pltpu.make_async_copy(main_hbm.at[idx_v[p]], rows_v[p], sem[p]).wait()
            pltpu.sync_copy(rows_v[p], out_hbm.at[pl.ds(r0, _K), pl.ds(0, _DM)])
            pltpu.sync_copy(tails_v[p], out_hbm.at[pl.ds(r0, _K), pl.ds(_DM, 1)])

        prep_start(0, 0)
        prep_start(1, 1)

        def pair(h, carry):
            g = 2 * h
            wait_store(g, 0)
            prep_start(g + 2, 0)
            wait_store(g + 1, 1)
            prep_start(g + 3, 1)
            return carry

        lax.fori_loop(0, _NCH // 2 - 1, pair, 0)
        wait_store(_NCH - 2, 0)
        wait_store(_NCH - 1, 1)

    return k(main, tail, xflat)


def kernel(x, token_table, ooba_table, pos_table):
    main, tail = _build_tables(token_table, ooba_table, pos_table)
    out = _sc_gather(main.reshape(_L * _V, _DM), tail.reshape(_L * _V),
                     x.reshape(-1).astype(jnp.int32))
    return out.reshape(_B, _L, _D)


# speedup vs baseline: 2.0142x; 2.0142x over previous
import functools

import jax
import jax.numpy as jnp
from jax import lax
from jax.experimental import pallas as pl
from jax.experimental.pallas import tpu as pltpu
from jax.experimental.pallas import tpu_sc as plsc

_B, _L, _V, _D = 4096, 200, 32, 129
_NW = 32
_BC = 128
_NBT = _B // _BC


def _build_table(token_table, ooba_table, pos_table):
    def body(tok_ref, ooba_ref, pos_ref, tbl_ref):
        comb = jnp.concatenate([tok_ref[...], ooba_ref[...]], axis=-1)
        tbl_ref[...] = comb.T[None, :, :] + pos_ref[...][:, :, None]

    return pl.pallas_call(
        body,
        out_shape=jax.ShapeDtypeStruct((_L, _D, _V), jnp.float32),
    )(token_table, ooba_table, pos_table)


def _sc_lookup(tbl, xt):
    mesh = plsc.VectorSubcoreMesh(core_axis_name="c", subcore_axis_name="s")

    @functools.partial(
        pl.kernel,
        mesh=mesh,
        out_type=jax.ShapeDtypeStruct((_D, _L, _B), jnp.float32),
        scratch_types=[
            pltpu.VMEM((_D * _V,), jnp.float32),
            pltpu.VMEM((_B,), jnp.int32),
            [pltpu.VMEM((_D, 1, _BC), jnp.float32)] * 2,
            [pltpu.SemaphoreType.DMA] * 2,
        ],
        compiler_params=pltpu.CompilerParams(needs_layout_passes=False),
    )
    def k(tbl_hbm, xt_hbm, out_hbm, tbl_v, xv, buf, sem):
        wid = lax.axis_index("s") * 2 + lax.axis_index("c")
        l_lo = wid * 6 + jnp.minimum(wid, 8)
        l_hi = l_lo + 6 + jnp.where(wid < 8, 1, 0)

        def fill(bt, p):
            xs = [xv[pl.ds(bt * _BC + j * 16, 16)] for j in range(_BC // 16)]

            def dbody(d, c):
                for j in range(_BC // 16):
                    vals = plsc.load_gather(tbl_v, [xs[j] + d * _V])
                    buf[p][d, 0, pl.ds(j * 16, 16)] = vals
                return c

            lax.fori_loop(0, _D, dbody, 0)

        def flush_start(l, bt, p):
            pltpu.async_copy(
                buf[p],
                out_hbm.at[pl.ds(0, _D), pl.ds(l, 1), pl.ds(bt * _BC, _BC)],
                sem[p])

        def flush_wait(l, bt, p):
            pltpu.make_async_copy(
                buf[p],
                out_hbm.at[pl.ds(0, _D), pl.ds(l, 1), pl.ds(bt * _BC, _BC)],
                sem[p]).wait()

        def per_l(l, carry):
            pltpu.sync_copy(tbl_hbm.at[pl.ds(l * _D * _V, _D * _V)], tbl_v)
            pltpu.sync_copy(xt_hbm.at[pl.ds(l * _B, _B)], xv)
            fill(0, 0)
            flush_start(l, 0, 0)
            fill(1, 1)
            flush_start(l, 1, 1)

            def pair(h, c):
                bt = 2 * h
                flush_wait(l, bt, 0)
                fill(bt + 2, 0)
                flush_start(l, bt + 2, 0)
                flush_wait(l, bt + 1, 1)
                fill(bt + 3, 1)
                flush_start(l, bt + 3, 1)
                return c

            lax.fori_loop(0, _NBT // 2 - 1, pair, 0)
            flush_wait(l, _NBT - 2, 0)
            flush_wait(l, _NBT - 1, 1)
            return carry

        lax.fori_loop(l_lo, l_hi, per_l, 0)

    return k(tbl, xt)


def kernel(x, token_table, ooba_table, pos_table):
    tbl = _build_table(token_table, ooba_table, pos_table).reshape(-1)
    xt = x.T.reshape(-1).astype(jnp.int32)
    out = _sc_lookup(tbl, xt)
    return out.transpose(2, 1, 0)


# speedup vs baseline: 7.6597x; 3.8029x over previous
import functools

import jax
import jax.numpy as jnp
from jax import lax
from jax.experimental import pallas as pl
from jax.experimental.pallas import tpu as pltpu
from jax.experimental.pallas import tpu_sc as plsc

_B, _L, _V, _D = 4096, 200, 32, 129
_NW = 32
_BC = 128
_NBT = _B // _BC


def _build_table(token_table, ooba_table, pos_table):
    def body(tok_ref, ooba_ref, pos_ref, tbl_ref):
        comb = jnp.concatenate([tok_ref[...], ooba_ref[...]], axis=-1)
        tbl_ref[...] = comb.T[None, :, :] + pos_ref[...][:, :, None]

    return pl.pallas_call(
        body,
        out_shape=jax.ShapeDtypeStruct((_L, _D, _V), jnp.float32),
    )(token_table, ooba_table, pos_table)


def _sc_lookup(tbl, xt):
    mesh = plsc.VectorSubcoreMesh(core_axis_name="c", subcore_axis_name="s")

    @functools.partial(
        pl.kernel,
        mesh=mesh,
        out_type=jax.ShapeDtypeStruct((_D, _L, _B), jnp.float32),
        scratch_types=[
            pltpu.VMEM((_D * _V,), jnp.float32),
            pltpu.VMEM((_B,), jnp.int32),
            [pltpu.VMEM((_D, 1, _BC), jnp.float32)] * 2,
            [pltpu.SemaphoreType.DMA] * 2,
        ],
        compiler_params=pltpu.CompilerParams(needs_layout_passes=False),
    )
    def k(tbl_hbm, xt_hbm, out_hbm, tbl_v, xv, buf, sem):
        wid = lax.axis_index("s") * 2 + lax.axis_index("c")
        l_lo = wid * 6 + jnp.minimum(wid, 8)
        l_hi = l_lo + 6 + jnp.where(wid < 8, 1, 0)

        def fill(bt, p):
            xs = [xv[pl.ds(bt * _BC + j * 16, 16)] for j in range(_BC // 16)]

            @plsc.parallel_loop(0, _D, unroll=4)
            def dbody(d):
                for j in range(_BC // 16):
                    vals = plsc.load_gather(tbl_v, [xs[j] + d * _V])
                    buf[p][d, 0, pl.ds(j * 16, 16)] = vals

        def flush_start(l, bt, p):
            pltpu.async_copy(
                buf[p],
                out_hbm.at[pl.ds(0, _D), pl.ds(l, 1), pl.ds(bt * _BC, _BC)],
                sem[p])

        def flush_wait(l, bt, p):
            pltpu.make_async_copy(
                buf[p],
                out_hbm.at[pl.ds(0, _D), pl.ds(l, 1), pl.ds(bt * _BC, _BC)],
                sem[p]).wait()

        def per_l(l, carry):
            pltpu.sync_copy(tbl_hbm.at[pl.ds(l * _D * _V, _D * _V)], tbl_v)
            pltpu.sync_copy(xt_hbm.at[pl.ds(l * _B, _B)], xv)
            fill(0, 0)
            flush_start(l, 0, 0)
            fill(1, 1)
            flush_start(l, 1, 1)

            def pair(h, c):
                bt = 2 * h
                flush_wait(l, bt, 0)
                fill(bt + 2, 0)
                flush_start(l, bt + 2, 0)
                flush_wait(l, bt + 1, 1)
                fill(bt + 3, 1)
                flush_start(l, bt + 3, 1)
                return c

            lax.fori_loop(0, _NBT // 2 - 1, pair, 0)
            flush_wait(l, _NBT - 2, 0)
            flush_wait(l, _NBT - 1, 1)
            return carry

        lax.fori_loop(l_lo, l_hi, per_l, 0)

    return k(tbl, xt)


def kernel(x, token_table, ooba_table, pos_table):
    tbl = _build_table(token_table, ooba_table, pos_table).reshape(-1)
    xt = x.T.reshape(-1).astype(jnp.int32)
    out = _sc_lookup(tbl, xt)
    return out.transpose(2, 1, 0)
